# baseline (speedup 1.0000x reference)
import functools

import jax
import jax.numpy as jnp
from jax import lax
from jax.experimental import pallas as pl
from jax.experimental.pallas import tpu as pltpu
from jax.experimental.pallas import tpu_sc as plsc

N = 10000
E = 320000
D_FEAT = 128
HID = 128
NCLS = 40
N_ITER = 10
LEAKY = 0.2

NW = 32
CH = 128
K = 80
E_PAD = NW * K * CH
NACC = 10240
RPT = NACC // 16
BLK = 1000
GRID = N // BLK


CHS = 128
KS = E_PAD // (NW * CHS)
NB = 2
GA = 1
NBI = 8
PF = 4


def _make_segsum(dw: int):
    mesh = plsc.VectorSubcoreMesh(core_axis_name="c", subcore_axis_name="s")

    @functools.partial(
        pl.kernel,
        mesh=mesh,
        out_type=jax.ShapeDtypeStruct((2, NACC, dw), jnp.float32),
        scratch_types=[
            pltpu.VMEM((NBI, 2, CHS), jnp.int32),
        ] + [pltpu.VMEM((CHS, dw), jnp.float32) for _ in range(NB)] + [
            pltpu.VMEM_SHARED((NACC, dw), jnp.float32),
        ] + [pltpu.SemaphoreType.DMA for _ in range(2 * NB + NBI)],
    )
    def seg(vals, eidx, zrows, out, ring, *rest):
        bufs = rest[:NB]
        acc = rest[NB]
        gsem = rest[NB + 1:NB + 1 + NB]
        ssem = rest[NB + 1 + NB:NB + 1 + 2 * NB]
        isem = rest[NB + 1 + 2 * NB:]
        c = lax.axis_index("c")
        s = lax.axis_index("s")
        wid = s * 2 + c

        def wait_data(sem, b):
            pltpu.make_async_copy(vals.at[pl.ds(0, CHS)], bufs[b], sem).wait()

        def wait_idx(slot):
            pltpu.make_async_copy(eidx.at[wid, 0], ring.at[slot],
                                  isem[slot]).wait()

        for z in range(RPT // CHS):
            pltpu.async_copy(zrows.at[pl.ds(0, CHS)],
                             acc.at[pl.ds(s * RPT + z * CHS, CHS)], ssem[0])
        for z in range(RPT // CHS):
            pltpu.make_async_copy(
                zrows.at[pl.ds(0, CHS)],
                acc.at[pl.ds(s * RPT + z * CHS, CHS)], ssem[0]).wait()
        plsc.subcore_barrier()

        for slot in range(PF):
            pltpu.async_copy(eidx.at[wid, slot], ring.at[slot], isem[slot])
        for m in range(GA):
            wait_idx(m)
            pltpu.async_copy(vals.at[ring.at[m, 0]], bufs[m], gsem[m])

        def round_body(r, carry):
            for u in range(NBI):
                j = r * NBI + u
                b = u % NB
                wait_data(gsem[b], b)
                pltpu.async_copy(bufs[b], acc.at[ring.at[u, 1]], ssem[b],
                                 add=True)
                m = j + GA
                bm = (u + GA) % NB
                um = (u + GA) % NBI

                @pl.when(m < KS)
                def _():
                    @pl.when(m >= NB)
                    def _():
                        wait_data(ssem[bm], bm)
                    wait_idx(um)
                    pltpu.async_copy(vals.at[ring.at[um, 0]], bufs[bm],
                                     gsem[bm])

                m2 = j + PF
                u2 = (u + PF) % NBI

                @pl.when(m2 < KS)
                def _():
                    pltpu.async_copy(eidx.at[wid, m2], ring.at[u2],
                                     isem[u2])
            return carry

        lax.fori_loop(0, KS // NBI, round_body, 0)
        for b in range(NB):
            wait_data(ssem[b], b)
        plsc.subcore_barrier()
        for z in range(RPT // CHS):
            sl = pl.ds(s * RPT + z * CHS, CHS)
            pltpu.async_copy(acc.at[sl], out.at[c, sl], gsem[0])
        for z in range(RPT // CHS):
            sl = pl.ds(s * RPT + z * CHS, CHS)
            pltpu.make_async_copy(acc.at[sl], out.at[c, sl], gsem[0]).wait()

    return seg


_segsum128 = _make_segsum(HID)


NBI_H = 8
NSS_H = 4
PF_H = 4


def _make_hist():
    mesh = plsc.VectorSubcoreMesh(core_axis_name="c", subcore_axis_name="s")

    @functools.partial(
        pl.kernel,
        mesh=mesh,
        out_type=jax.ShapeDtypeStruct((2, NACC, HID), jnp.float32),
        scratch_types=[
            pltpu.VMEM((NBI_H, CH), jnp.int32),
            pltpu.VMEM((CH, HID), jnp.float32),
            pltpu.VMEM_SHARED((NACC, HID), jnp.float32),
        ] + [pltpu.SemaphoreType.DMA for _ in range(NSS_H + NBI_H)],
    )
    def hist(zrows, ones_rows, cidx, out, ring, buf, acc, *sems):
        ssem = sems[:NSS_H]
        isem = sems[NSS_H:]
        c = lax.axis_index("c")
        s = lax.axis_index("s")
        wid = s * 2 + c

        def wait_idx(slot):
            pltpu.make_async_copy(cidx.at[wid, 0], ring.at[slot],
                                  isem[slot]).wait()

        def wait_sc(b):
            pltpu.make_async_copy(buf, acc.at[pl.ds(0, CH)], ssem[b]).wait()

        for z in range(RPT // CH):
            pltpu.async_copy(zrows,
                             acc.at[pl.ds(s * RPT + z * CH, CH)], ssem[0])
        pltpu.sync_copy(ones_rows, buf)
        for z in range(RPT // CH):
            pltpu.make_async_copy(
                zrows, acc.at[pl.ds(s * RPT + z * CH, CH)], ssem[0]).wait()
        plsc.subcore_barrier()

        for slot in range(PF_H):
            pltpu.async_copy(cidx.at[wid, slot], ring.at[slot], isem[slot])

        def round_body(r, carry):
            for u in range(NBI_H):
                j = r * NBI_H + u
                wait_idx(u)

                @pl.when(j >= NSS_H)
                def _():
                    wait_sc(u % NSS_H)

                pltpu.async_copy(buf, acc.at[ring.at[u]], ssem[u % NSS_H],
                                 add=True)
                m = j + PF_H
                um = (u + PF_H) % NBI_H

                @pl.when(m < K)
                def _():
                    pltpu.async_copy(cidx.at[wid, m], ring.at[um], isem[um])
            return carry

        lax.fori_loop(0, K // NBI_H, round_body, 0)
        for b in range(NSS_H):
            wait_sc(b)
        plsc.subcore_barrier()
        for z in range(RPT // CH):
            sl = pl.ds(s * RPT + z * CH, CH)
            pltpu.async_copy(acc.at[sl], out.at[c, sl], ssem[0])
        for z in range(RPT // CH):
            sl = pl.ds(s * RPT + z * CH, CH)
            pltpu.make_async_copy(acc.at[sl], out.at[c, sl], ssem[0]).wait()

    return hist


_hist = _make_hist()


def _prep_body(x_ref, wi_ref, bi_ref, out_ref):
    out_ref[...] = lax.dot_general(
        x_ref[...], wi_ref[...], (((1,), (1,)), ((), ())),
        preferred_element_type=jnp.float32) + bi_ref[...]


def _iter0_body(part_ref, inf_ref, wh_ref, bh_ref, s2n_ref):
    p = part_ref[0] + part_ref[1]
    lane = lax.broadcasted_iota(jnp.int32, p.shape, 1)
    cnt = jnp.sum(jnp.where(lane == 0, p, 0.0), axis=1, keepdims=True)
    bh = bh_ref[...]
    post = jnp.tanh(inf_ref[...] + cnt * bh)
    st = (1.0 - LEAKY) * bh + LEAKY * post
    s2n_ref[...] = lax.dot_general(
        st, wh_ref[...], (((1,), (1,)), ((), ())),
        preferred_element_type=jnp.float32) + bh


def _iter_body(s2_ref, part_ref, inf_ref, wh_ref, bh_ref, st_ref, s2n_ref):
    neigh = part_ref[0] + part_ref[1]
    post = jnp.tanh(inf_ref[...] + neigh)
    st = (1.0 - LEAKY) * s2_ref[...] + LEAKY * post
    st_ref[...] = st
    s2n_ref[...] = lax.dot_general(
        st, wh_ref[...], (((1,), (1,)), ((), ())),
        preferred_element_type=jnp.float32) + bh_ref[...]


def _readout_body(st_ref, p0_ref, wg_ref, emb_ref, g_ref, aux_ref):
    emb = jnp.maximum(st_ref[...], 0.0)
    emb_ref[...] = emb
    h = lax.dot_general(emb, wg_ref[...], (((1,), (1,)), ((), ())),
                        preferred_element_type=jnp.float32)
    p = p0_ref[0] + p0_ref[1]
    lane128 = lax.broadcasted_iota(jnp.int32, p.shape, 1)
    cnt = jnp.sum(jnp.where(lane128 == 0, p, 0.0), axis=1, keepdims=True)
    deg = cnt + 1.0
    dis = lax.rsqrt(deg)
    g_ref[...] = h * dis
    lane = lax.broadcasted_iota(jnp.int32, h.shape, 1)
    aux_ref[...] = jnp.where(lane == NCLS, dis, h / deg)


def _final_body(pr_ref, aux_ref, bg_ref, logp_ref):
    seg = pr_ref[0] + pr_ref[1]
    aux = aux_ref[...]
    lane = lax.broadcasted_iota(jnp.int32, aux.shape, 1)
    dis = jnp.sum(jnp.where(lane == NCLS, aux, 0.0), axis=1, keepdims=True)
    selfterm = jnp.where(lane < NCLS, aux, 0.0)
    out = dis * seg + selfterm + bg_ref[...]
    masked = jnp.where(lane < NCLS, out, -1e30)
    m = jnp.max(masked, axis=1, keepdims=True)
    ssum = jnp.sum(jnp.exp(masked - m), axis=1, keepdims=True)
    logp_ref[...] = out - m - jnp.log(ssum)


def _row_spec(dw):
    return pl.BlockSpec((BLK, dw), lambda i: (i, 0))


def _full_spec(shape):
    nd = len(shape)
    return pl.BlockSpec(shape, lambda i: (0,) * nd)


def _part_spec(dw):
    return pl.BlockSpec((2, BLK, dw), lambda i: (0, i, 0))


def _f32(shape):
    return jax.ShapeDtypeStruct(shape, jnp.float32)


def kernel(x, edge_index, Wi, bi, Wh, bh, Wg, bg):
    row = edge_index[0]
    col = edge_index[1]
    pad = E_PAD - E
    ridx = jnp.concatenate([row, jnp.zeros((pad,), jnp.int32)]).reshape(NW, K, CH)
    cidx = jnp.concatenate([col, jnp.full((pad,), N, jnp.int32)]).reshape(NW, K, CH)
    eidx = jnp.stack([ridx.reshape(NW, KS, CHS), cidx.reshape(NW, KS, CHS)],
                     axis=2)
    z128 = jnp.zeros((CH, HID), jnp.float32)
    bi2 = bi.reshape(1, HID)
    bh2 = bh.reshape(1, HID)
    bg2 = jnp.pad(bg, (0, 128 - NCLS)).reshape(1, 128)
    wg_pad = jnp.pad(Wg, ((0, 128 - NCLS), (0, 0)))

    input_feat = pl.pallas_call(
        _prep_body, grid=(GRID,),
        in_specs=[_row_spec(D_FEAT), _full_spec((HID, D_FEAT)), _full_spec((1, HID))],
        out_specs=_row_spec(HID),
        out_shape=_f32((N, HID)),
    )(x, Wi, bi2)

    ones128 = jnp.ones((CH, HID), jnp.float32)
    p0 = _hist(z128, ones128, cidx)

    s2 = pl.pallas_call(
        _iter0_body, grid=(GRID,),
        in_specs=[_part_spec(HID), _row_spec(HID),
                  _full_spec((HID, HID)), _full_spec((1, HID))],
        out_specs=_row_spec(HID),
        out_shape=_f32((N, HID)),
    )(p0, input_feat, Wh, bh2)

    iter_call = pl.pallas_call(
        _iter_body, grid=(GRID,),
        in_specs=[_row_spec(HID), _part_spec(HID), _row_spec(HID),
                  _full_spec((HID, HID)), _full_spec((1, HID))],
        out_specs=[_row_spec(HID), _row_spec(HID)],
        out_shape=[_f32((N, HID)), _f32((N, HID))],
    )
    st = None
    for _ in range(1, N_ITER):
        part = _segsum128(s2, eidx, z128)
        st, s2 = iter_call(s2, part, input_feat, Wh, bh2)

    node_emb, g_pad, aux = pl.pallas_call(
        _readout_body, grid=(GRID,),
        in_specs=[_row_spec(HID), _part_spec(HID), _full_spec((128, HID))],
        out_specs=[_row_spec(HID), _row_spec(128), _row_spec(128)],
        out_shape=[_f32((N, HID)), _f32((N, 128)), _f32((N, 128))],
    )(st, p0, wg_pad)

    pr = _segsum128(g_pad, eidx, z128)

    logp_pad = pl.pallas_call(
        _final_body, grid=(GRID,),
        in_specs=[_part_spec(128), _row_spec(128), _full_spec((1, 128))],
        out_specs=_row_spec(128),
        out_shape=_f32((N, 128)),
    )(pr, aux, bg2)

    return (logp_pad[:, :NCLS], node_emb)


# speedup vs baseline: 1.0265x; 1.0265x over previous
import functools

import jax
import jax.numpy as jnp
from jax import lax
from jax.experimental import pallas as pl
from jax.experimental.pallas import tpu as pltpu
from jax.experimental.pallas import tpu_sc as plsc

N = 10000
E = 320000
D_FEAT = 128
HID = 128
NCLS = 40
N_ITER = 10
LEAKY = 0.2

NW = 32
CH = 128
K = 80
E_PAD = NW * K * CH
NACC = 10240
RPT = NACC // 16
BLK = 1000
GRID = N // BLK


CHS = 128
KS = E_PAD // (NW * CHS)
NB = 2
GA = 1
NBI = 8
PF = 4


def _make_segsum(dw: int):
    mesh = plsc.VectorSubcoreMesh(core_axis_name="c", subcore_axis_name="s")

    @functools.partial(
        pl.kernel,
        mesh=mesh,
        out_type=jax.ShapeDtypeStruct((2, NACC, dw), jnp.float32),
        scratch_types=[
            pltpu.VMEM((NBI, 2, CHS), jnp.int32),
        ] + [pltpu.VMEM((CHS, dw), jnp.float32) for _ in range(NB)] + [
            pltpu.VMEM_SHARED((NACC, dw), jnp.float32),
        ] + [pltpu.SemaphoreType.DMA for _ in range(2 * NB + NBI)],
    )
    def seg(vals, eidx, zrows, out, ring, *rest):
        bufs = rest[:NB]
        acc = rest[NB]
        gsem = rest[NB + 1:NB + 1 + NB]
        ssem = rest[NB + 1 + NB:NB + 1 + 2 * NB]
        isem = rest[NB + 1 + 2 * NB:]
        c = lax.axis_index("c")
        s = lax.axis_index("s")
        wid = s * 2 + c

        def wait_data(sem, b):
            pltpu.make_async_copy(vals.at[pl.ds(0, CHS)], bufs[b], sem).wait()

        def wait_idx(slot):
            pltpu.make_async_copy(eidx.at[wid, 0], ring.at[slot],
                                  isem[slot]).wait()

        pltpu.sync_copy(zrows.at[pl.ds(0, CHS)], bufs[0])
        for z in range(RPT // CHS):
            pltpu.async_copy(bufs[0],
                             acc.at[pl.ds(s * RPT + z * CHS, CHS)], ssem[0])
        for z in range(RPT // CHS):
            pltpu.make_async_copy(
                bufs[0],
                acc.at[pl.ds(s * RPT + z * CHS, CHS)], ssem[0]).wait()
        plsc.subcore_barrier()

        for slot in range(PF):
            pltpu.async_copy(eidx.at[wid, slot], ring.at[slot], isem[slot])
        for m in range(GA):
            wait_idx(m)
            pltpu.async_copy(vals.at[ring.at[m, 0]], bufs[m], gsem[m])

        def round_body(r, carry):
            for u in range(NBI):
                j = r * NBI + u
                b = u % NB
                wait_data(gsem[b], b)
                pltpu.async_copy(bufs[b], acc.at[ring.at[u, 1]], ssem[b],
                                 add=True)
                m = j + GA
                bm = (u + GA) % NB
                um = (u + GA) % NBI

                @pl.when(m < KS)
                def _():
                    @pl.when(m >= NB)
                    def _():
                        wait_data(ssem[bm], bm)
                    wait_idx(um)
                    pltpu.async_copy(vals.at[ring.at[um, 0]], bufs[bm],
                                     gsem[bm])

                m2 = j + PF
                u2 = (u + PF) % NBI

                @pl.when(m2 < KS)
                def _():
                    pltpu.async_copy(eidx.at[wid, m2], ring.at[u2],
                                     isem[u2])
            return carry

        lax.fori_loop(0, KS // NBI, round_body, 0)
        for b in range(NB):
            wait_data(ssem[b], b)
        plsc.subcore_barrier()
        for z in range(RPT // CHS):
            sl = pl.ds(s * RPT + z * CHS, CHS)
            b = z % NB

            @pl.when(z >= NB)
            def _():
                pltpu.make_async_copy(bufs[b], out.at[c, sl], gsem[b]).wait()

            pltpu.sync_copy(acc.at[sl], bufs[b])
            pltpu.async_copy(bufs[b], out.at[c, sl], gsem[b])
        for b in range(NB):
            sl = pl.ds(0, CHS)
            pltpu.make_async_copy(bufs[b], out.at[c, sl], gsem[b]).wait()

    return seg


_segsum128 = _make_segsum(HID)


NBI_H = 8
NSS_H = 4
PF_H = 4


def _make_hist():
    mesh = plsc.VectorSubcoreMesh(core_axis_name="c", subcore_axis_name="s")

    @functools.partial(
        pl.kernel,
        mesh=mesh,
        out_type=jax.ShapeDtypeStruct((2, NACC, HID), jnp.float32),
        scratch_types=[
            pltpu.VMEM((NBI_H, CH), jnp.int32),
            pltpu.VMEM((CH, HID), jnp.float32),
            pltpu.VMEM_SHARED((NACC, HID), jnp.float32),
        ] + [pltpu.SemaphoreType.DMA for _ in range(NSS_H + NBI_H)],
    )
    def hist(zrows, ones_rows, cidx, out, ring, buf, acc, *sems):
        ssem = sems[:NSS_H]
        isem = sems[NSS_H:]
        c = lax.axis_index("c")
        s = lax.axis_index("s")
        wid = s * 2 + c

        def wait_idx(slot):
            pltpu.make_async_copy(cidx.at[wid, 0], ring.at[slot],
                                  isem[slot]).wait()

        def wait_sc(b):
            pltpu.make_async_copy(buf, acc.at[pl.ds(0, CH)], ssem[b]).wait()

        for z in range(RPT // CH):
            pltpu.async_copy(zrows,
                             acc.at[pl.ds(s * RPT + z * CH, CH)], ssem[0])
        pltpu.sync_copy(ones_rows, buf)
        for z in range(RPT // CH):
            pltpu.make_async_copy(
                zrows, acc.at[pl.ds(s * RPT + z * CH, CH)], ssem[0]).wait()
        plsc.subcore_barrier()

        for slot in range(PF_H):
            pltpu.async_copy(cidx.at[wid, slot], ring.at[slot], isem[slot])

        def round_body(r, carry):
            for u in range(NBI_H):
                j = r * NBI_H + u
                wait_idx(u)

                @pl.when(j >= NSS_H)
                def _():
                    wait_sc(u % NSS_H)

                pltpu.async_copy(buf, acc.at[ring.at[u]], ssem[u % NSS_H],
                                 add=True)
                m = j + PF_H
                um = (u + PF_H) % NBI_H

                @pl.when(m < K)
                def _():
                    pltpu.async_copy(cidx.at[wid, m], ring.at[um], isem[um])
            return carry

        lax.fori_loop(0, K // NBI_H, round_body, 0)
        for b in range(NSS_H):
            wait_sc(b)
        plsc.subcore_barrier()
        for z in range(RPT // CH):
            sl = pl.ds(s * RPT + z * CH, CH)
            pltpu.async_copy(acc.at[sl], out.at[c, sl], ssem[0])
        for z in range(RPT // CH):
            sl = pl.ds(s * RPT + z * CH, CH)
            pltpu.make_async_copy(acc.at[sl], out.at[c, sl], ssem[0]).wait()

    return hist


_hist = _make_hist()


def _prep_body(x_ref, wi_ref, bi_ref, out_ref):
    out_ref[...] = lax.dot_general(
        x_ref[...], wi_ref[...], (((1,), (1,)), ((), ())),
        preferred_element_type=jnp.float32) + bi_ref[...]


def _iter0_body(part_ref, inf_ref, wh_ref, bh_ref, s2n_ref):
    p = part_ref[0] + part_ref[1]
    lane = lax.broadcasted_iota(jnp.int32, p.shape, 1)
    cnt = jnp.sum(jnp.where(lane == 0, p, 0.0), axis=1, keepdims=True)
    bh = bh_ref[...]
    post = jnp.tanh(inf_ref[...] + cnt * bh)
    st = (1.0 - LEAKY) * bh + LEAKY * post
    s2n_ref[...] = lax.dot_general(
        st, wh_ref[...], (((1,), (1,)), ((), ())),
        preferred_element_type=jnp.float32) + bh


def _iter_body(s2_ref, part_ref, inf_ref, wh_ref, bh_ref, st_ref, s2n_ref):
    neigh = part_ref[0] + part_ref[1]
    post = jnp.tanh(inf_ref[...] + neigh)
    st = (1.0 - LEAKY) * s2_ref[...] + LEAKY * post
    st_ref[...] = st
    s2n_ref[...] = lax.dot_general(
        st, wh_ref[...], (((1,), (1,)), ((), ())),
        preferred_element_type=jnp.float32) + bh_ref[...]


def _readout_body(st_ref, p0_ref, wg_ref, emb_ref, g_ref, aux_ref):
    emb = jnp.maximum(st_ref[...], 0.0)
    emb_ref[...] = emb
    h = lax.dot_general(emb, wg_ref[...], (((1,), (1,)), ((), ())),
                        preferred_element_type=jnp.float32)
    p = p0_ref[0] + p0_ref[1]
    lane128 = lax.broadcasted_iota(jnp.int32, p.shape, 1)
    cnt = jnp.sum(jnp.where(lane128 == 0, p, 0.0), axis=1, keepdims=True)
    deg = cnt + 1.0
    dis = lax.rsqrt(deg)
    g_ref[...] = h * dis
    lane = lax.broadcasted_iota(jnp.int32, h.shape, 1)
    aux_ref[...] = jnp.where(lane == NCLS, dis, h / deg)


def _final_body(pr_ref, aux_ref, bg_ref, logp_ref):
    seg = pr_ref[0] + pr_ref[1]
    aux = aux_ref[...]
    lane = lax.broadcasted_iota(jnp.int32, aux.shape, 1)
    dis = jnp.sum(jnp.where(lane == NCLS, aux, 0.0), axis=1, keepdims=True)
    selfterm = jnp.where(lane < NCLS, aux, 0.0)
    out = dis * seg + selfterm + bg_ref[...]
    masked = jnp.where(lane < NCLS, out, -1e30)
    m = jnp.max(masked, axis=1, keepdims=True)
    ssum = jnp.sum(jnp.exp(masked - m), axis=1, keepdims=True)
    logp_ref[...] = out - m - jnp.log(ssum)


def _row_spec(dw):
    return pl.BlockSpec((BLK, dw), lambda i: (i, 0))


def _full_spec(shape):
    nd = len(shape)
    return pl.BlockSpec(shape, lambda i: (0,) * nd)


def _part_spec(dw):
    return pl.BlockSpec((2, BLK, dw), lambda i: (0, i, 0))


def _f32(shape):
    return jax.ShapeDtypeStruct(shape, jnp.float32)


def kernel(x, edge_index, Wi, bi, Wh, bh, Wg, bg):
    row = edge_index[0]
    col = edge_index[1]
    pad = E_PAD - E
    ridx = jnp.concatenate([row, jnp.zeros((pad,), jnp.int32)]).reshape(NW, K, CH)
    cidx = jnp.concatenate([col, jnp.full((pad,), N, jnp.int32)]).reshape(NW, K, CH)
    eidx = jnp.stack([ridx.reshape(NW, KS, CHS), cidx.reshape(NW, KS, CHS)],
                     axis=2)
    z128 = jnp.zeros((CH, HID), jnp.float32)
    bi2 = bi.reshape(1, HID)
    bh2 = bh.reshape(1, HID)
    bg2 = jnp.pad(bg, (0, 128 - NCLS)).reshape(1, 128)
    wg_pad = jnp.pad(Wg, ((0, 128 - NCLS), (0, 0)))

    input_feat = pl.pallas_call(
        _prep_body, grid=(GRID,),
        in_specs=[_row_spec(D_FEAT), _full_spec((HID, D_FEAT)), _full_spec((1, HID))],
        out_specs=_row_spec(HID),
        out_shape=_f32((N, HID)),
    )(x, Wi, bi2)

    ones128 = jnp.ones((CH, HID), jnp.float32)
    p0 = _hist(z128, ones128, cidx)

    s2 = pl.pallas_call(
        _iter0_body, grid=(GRID,),
        in_specs=[_part_spec(HID), _row_spec(HID),
                  _full_spec((HID, HID)), _full_spec((1, HID))],
        out_specs=_row_spec(HID),
        out_shape=_f32((N, HID)),
    )(p0, input_feat, Wh, bh2)

    iter_call = pl.pallas_call(
        _iter_body, grid=(GRID,),
        in_specs=[_row_spec(HID), _part_spec(HID), _row_spec(HID),
                  _full_spec((HID, HID)), _full_spec((1, HID))],
        out_specs=[_row_spec(HID), _row_spec(HID)],
        out_shape=[_f32((N, HID)), _f32((N, HID))],
    )
    st = None
    for _ in range(1, N_ITER):
        part = _segsum128(s2, eidx, z128)
        st, s2 = iter_call(s2, part, input_feat, Wh, bh2)

    node_emb, g_pad, aux = pl.pallas_call(
        _readout_body, grid=(GRID,),
        in_specs=[_row_spec(HID), _part_spec(HID), _full_spec((128, HID))],
        out_specs=[_row_spec(HID), _row_spec(128), _row_spec(128)],
        out_shape=[_f32((N, HID)), _f32((N, 128)), _f32((N, 128))],
    )(st, p0, wg_pad)

    pr = _segsum128(g_pad, eidx, z128)

    logp_pad = pl.pallas_call(
        _final_body, grid=(GRID,),
        in_specs=[_part_spec(128), _row_spec(128), _full_spec((1, 128))],
        out_specs=_row_spec(128),
        out_shape=_f32((N, 128)),
    )(pr, aux, bg2)

    return (logp_pad[:, :NCLS], node_emb)
